# baseline (speedup 1.0000x reference)
import dataclasses
import functools

import jax
import jax.numpy as jnp
from jax import lax
from jax.experimental import pallas as pl
from jax.experimental.pallas import tpu as pltpu
from jax.experimental.pallas import tpu_sc as plsc

N = 10000
E = 320000
D = 128
H = 128

NC = 2
NS = 16
NW = NC * NS
E_PAD = 327680
IDX_ROWS = E_PAD // 128
SPLIT0 = 152
SPLIT1 = (IDX_ROWS // NS) - SPLIT0
N_ACC = 10240
ROWS_PER_TILE = N_ACC // NS


def _mesh():
    return plsc.VectorSubcoreMesh(core_axis_name="c", subcore_axis_name="s",
                                  num_cores=2)


def _sc_params():
    cp = pltpu.CompilerParams()
    if "needs_layout_passes" in pltpu.CompilerParams.__dataclass_fields__:
        cp = dataclasses.replace(cp, needs_layout_passes=False)
    return cp


def _zero_fill(buf, nrows, ncol16):
    @pl.loop(0, nrows)
    def _(r):
        for j in range(ncol16):
            buf[r, pl.ds(j * 16, 16)] = jnp.zeros((16,), jnp.float32)


def _sc_agg(with_deg):
    out_types = [jax.ShapeDtypeStruct((NC, N_ACC, D), jnp.float32)]
    scratch = [
        pltpu.VMEM((1, 128), jnp.int32),
        pltpu.VMEM((1, 128), jnp.int32),
        pltpu.VMEM((1, 128), jnp.int32),
        pltpu.VMEM((1, 128), jnp.int32),
        pltpu.VMEM((128, D), jnp.float32),
        pltpu.VMEM((128, D), jnp.float32),
        pltpu.VMEM((16, 128), jnp.float32),
        pltpu.VMEM_SHARED((N_ACC, D), jnp.float32),
        pltpu.SemaphoreType.DMA,
        pltpu.SemaphoreType.DMA,
        pltpu.SemaphoreType.DMA,
        pltpu.SemaphoreType.DMA,
    ]
    if with_deg:
        out_types.append(
            jax.ShapeDtypeStruct((NW, N_ACC // 128, 128), jnp.float32))
        scratch.append(pltpu.VMEM((N_ACC // 128, 128), jnp.float32))

    def body(h_hbm, src_hbm, dst_hbm, *refs):
        if with_deg:
            (agg_hbm, deg_hbm, si0, si1, di0, di1, rw0, rw1, zbuf, acc,
             smi0, smi1, smg0, smg1, degloc) = refs
        else:
            (agg_hbm, si0, si1, di0, di1, rw0, rw1, zbuf, acc,
             smi0, smi1, smg0, smg1) = refs
        c = lax.axis_index("c")
        s = lax.axis_index("s")
        sidx = [si0, si1]
        didx = [di0, di1]
        rows = [rw0, rw1]
        semi = [smi0, smi1]
        semg = [smg0, smg1]

        _zero_fill(zbuf, 16, 8)
        base = s * ROWS_PER_TILE
        @pl.loop(0, ROWS_PER_TILE // 16)
        def _(i):
            pltpu.sync_copy(zbuf, acc.at[pl.ds(base + i * 16, 16)])
        if with_deg:
            _zero_fill(degloc, N_ACC // 128, 8)
        plsc.subcore_barrier()

        ones16 = jnp.ones((16,), jnp.float32)

        def fire_idx(r, p):
            pltpu.async_copy(src_hbm.at[pl.ds(r, 1)], sidx[p], semi[p])
            pltpu.async_copy(dst_hbm.at[pl.ds(r, 1)], didx[p], semi[p])

        def wait_idx(r, p):
            pltpu.make_async_copy(src_hbm.at[pl.ds(r, 1)], sidx[p],
                                  semi[p]).wait()
            pltpu.make_async_copy(dst_hbm.at[pl.ds(r, 1)], didx[p],
                                  semi[p]).wait()

        def fire_gather(p):
            pltpu.async_copy(h_hbm.at[sidx[p].at[0]], rows[p], semg[p])

        def wait_gather(p):
            pltpu.make_async_copy(h_hbm.at[sidx[p].at[0]], rows[p],
                                  semg[p]).wait()

        def scatter(p):
            pltpu.sync_copy(rows[p], acc.at[didx[p].at[0]], add=True)
            if with_deg:
                for g in range(8):
                    dv = didx[p][0, pl.ds(g * 16, 16)]
                    plsc.addupdate_scatter(
                        degloc,
                        [lax.shift_right_logical(dv, 7),
                         lax.bitwise_and(dv, 127)],
                        ones16)

        def pipeline(nrows, wbase):
            fire_idx(wbase, 0)
            fire_idx(wbase + 1, 1)

            def chunk(u, p):
                wait_idx(wbase + u, p)
                fire_gather(p)
                @pl.when(u > 0)
                def _():
                    wait_gather(1 - p)
                    scatter(1 - p)
                    @pl.when(u < nrows - 1)
                    def _():
                        fire_idx(wbase + u + 1, 1 - p)

            @pl.loop(0, nrows, step=2)
            def _(t):
                chunk(t, 0)
                chunk(t + 1, 1)

            wait_gather(1)
            scatter(1)

        @pl.when(c == 0)
        def _():
            pipeline(SPLIT0, s * SPLIT0)

        @pl.when(c == 1)
        def _():
            pipeline(SPLIT1, NS * SPLIT0 + s * SPLIT1)

        plsc.subcore_barrier()
        pltpu.sync_copy(acc.at[pl.ds(base, ROWS_PER_TILE)],
                        agg_hbm.at[c].at[pl.ds(base, ROWS_PER_TILE)])
        if with_deg:
            pltpu.sync_copy(degloc, deg_hbm.at[c * NS + s])

    return pl.kernel(body, out_type=out_types, mesh=_mesh(),
                     scratch_types=scratch, compiler_params=_sc_params())


def _sc_gather2(tab_a, tab_b, idx_a, idx_b):
    def body(ta_hbm, tb_hbm, ia_hbm, ib_hbm, oa_hbm, ob_hbm,
             ba0, ba1, bb0, bb1, ra0, ra1, rb0, rb1,
             smi0, smi1, sma0, sma1, smb0, smb1, swa0, swa1, swb0, swb1):
        c = lax.axis_index("c")
        s = lax.axis_index("s")
        bia = [ba0, ba1]
        bib = [bb0, bb1]
        rowsa = [ra0, ra1]
        rowsb = [rb0, rb1]
        semi = [smi0, smi1]
        sema = [sma0, sma1]
        semb = [smb0, smb1]
        semwa = [swa0, swa1]
        semwb = [swb0, swb1]

        def fire_idx(r, p):
            pltpu.async_copy(ia_hbm.at[pl.ds(r, 1)], bia[p], semi[p])
            pltpu.async_copy(ib_hbm.at[pl.ds(r, 1)], bib[p], semi[p])

        def wait_idx(r, p):
            pltpu.make_async_copy(ia_hbm.at[pl.ds(r, 1)], bia[p],
                                  semi[p]).wait()
            pltpu.make_async_copy(ib_hbm.at[pl.ds(r, 1)], bib[p],
                                  semi[p]).wait()

        def fire_gather(p):
            pltpu.async_copy(ta_hbm.at[bia[p].at[0]], rowsa[p], sema[p])
            pltpu.async_copy(tb_hbm.at[bib[p].at[0]], rowsb[p], semb[p])

        def wait_gather(p):
            pltpu.make_async_copy(ta_hbm.at[bia[p].at[0]], rowsa[p],
                                  sema[p]).wait()
            pltpu.make_async_copy(tb_hbm.at[bib[p].at[0]], rowsb[p],
                                  semb[p]).wait()

        def fire_write(r, p):
            pltpu.async_copy(rowsa[p], oa_hbm.at[pl.ds(r * 128, 128)],
                             semwa[p])
            pltpu.async_copy(rowsb[p], ob_hbm.at[pl.ds(r * 128, 128)],
                             semwb[p])

        def wait_write(r, p):
            pltpu.make_async_copy(rowsa[p], oa_hbm.at[pl.ds(r * 128, 128)],
                                  semwa[p]).wait()
            pltpu.make_async_copy(rowsb[p], ob_hbm.at[pl.ds(r * 128, 128)],
                                  semwb[p]).wait()

        def pipeline(nrows, wbase):
            fire_idx(wbase, 0)
            fire_idx(wbase + 1, 1)

            def chunk(u, p):
                wait_idx(wbase + u, p)
                @pl.when(u > 1)
                def _():
                    wait_write(wbase + u - 2, p)
                fire_gather(p)
                @pl.when(u > 0)
                def _():
                    wait_gather(1 - p)
                    fire_write(wbase + u - 1, 1 - p)
                    @pl.when(u < nrows - 1)
                    def _():
                        fire_idx(wbase + u + 1, 1 - p)

            @pl.loop(0, nrows, step=2)
            def _(t):
                chunk(t, 0)
                chunk(t + 1, 1)

            wait_gather(1)
            fire_write(wbase + nrows - 1, 1)
            wait_write(wbase + nrows - 2, 0)
            wait_write(wbase + nrows - 1, 1)

        @pl.when(c == 0)
        def _():
            pipeline(SPLIT0, s * SPLIT0)

        @pl.when(c == 1)
        def _():
            pipeline(SPLIT1, NS * SPLIT0 + s * SPLIT1)

    k = pl.kernel(
        body,
        out_type=[jax.ShapeDtypeStruct((E_PAD, D), jnp.float32),
                  jax.ShapeDtypeStruct((E_PAD, D), jnp.float32)],
        mesh=_mesh(),
        scratch_types=(
            [pltpu.VMEM((1, 128), jnp.int32) for _ in range(4)]
            + [pltpu.VMEM((128, D), jnp.float32) for _ in range(4)]
            + [pltpu.SemaphoreType.DMA for _ in range(10)]
        ),
        compiler_params=_sc_params(),
    )
    return k(tab_a, tab_b, idx_a, idx_b)


def _tc_prelude(x, g, b, w1, b1, w2, b2):
    def body(x_ref, g_ref, b_ref, w1_ref, b1_ref, w2_ref, b2_ref, o_ref):
        xv = x_ref[...]
        mu = jnp.mean(xv, axis=0, keepdims=True)
        var = jnp.mean((xv - mu) ** 2, axis=0, keepdims=True)
        h = (xv - mu) / jnp.sqrt(var + 1e-5) * g_ref[...] + b_ref[...]
        h = jax.nn.gelu(jnp.dot(h, w1_ref[...],
                                preferred_element_type=jnp.float32)
                        + b1_ref[...])
        h = jnp.tanh(jnp.dot(h, w2_ref[...],
                             preferred_element_type=jnp.float32) + b2_ref[...])
        o_ref[...] = h
    return pl.pallas_call(
        body, out_shape=jax.ShapeDtypeStruct((N, D), jnp.float32),
    )(x, g, b, w1, b1, w2, b2)


def _tc_layer(h, agg, degp, ws, wn, bb, gw, gb, ga):
    def body(h_ref, a_ref, d_ref, ws_ref, wn_ref, bb_ref, gw_ref, gb_ref,
             ga_ref, o_ref):
        deg = lax.dot_general(
            d_ref[...], jnp.ones((NW, 1), jnp.float32),
            (((0,), (0,)), ((), ())),
            preferred_element_type=jnp.float32)[:N]
        deg = jnp.maximum(deg, 1.0)
        agg_v = (a_ref[0, :N, :] + a_ref[1, :N, :]) / deg
        hv = h_ref[...]
        h_new = (jnp.dot(hv, ws_ref[...], preferred_element_type=jnp.float32)
                 + jnp.dot(agg_v, wn_ref[...],
                           preferred_element_type=jnp.float32)
                 + bb_ref[...])
        m = jnp.mean(h_new, axis=0, keepdims=True)
        xc = h_new - ga_ref[0, 0] * m
        v = jnp.mean(xc ** 2, axis=0, keepdims=True)
        o_ref[...] = gw_ref[...] * xc / jnp.sqrt(v + 1e-5) + gb_ref[...]
    return pl.pallas_call(
        body, out_shape=jax.ShapeDtypeStruct((N, H), jnp.float32),
    )(h, agg, degp, ws, wn, bb, gw, gb, ga)


def _tc_final(xs0, xs1, xs2, x, dw, db, da, lw1, lb1, lw2, lb2, lw3, lb3):
    NB = 5
    BR = N // NB

    def body(x0_ref, x1_ref, x2_ref, x_ref, dw_ref, db_ref, da_ref,
             w1_ref, b1_ref, w2_ref, b2_ref, w3_ref, b3_ref, o_ref):
        hcat = jnp.concatenate(
            [x0_ref[...], x1_ref[...], x2_ref[...]], axis=1)
        t = dw_ref[...] * jnp.tanh(da_ref[0, 0] * hcat) + db_ref[...]
        t = jax.nn.gelu(jnp.dot(t, w1_ref[...],
                                preferred_element_type=jnp.float32)
                        + b1_ref[...])
        t = jax.nn.gelu(jnp.dot(t, w2_ref[...],
                                preferred_element_type=jnp.float32)
                        + b2_ref[...])
        t = jnp.dot(t, w3_ref[...],
                    preferred_element_type=jnp.float32) + b3_ref[...]
        o_ref[...] = t + x_ref[...]

    row_spec = pl.BlockSpec((BR, H), lambda i: (i, 0))
    full = lambda r, c: pl.BlockSpec((r, c), lambda i: (0, 0))
    return pl.pallas_call(
        body,
        grid=(NB,),
        in_specs=[row_spec, row_spec, row_spec, row_spec,
                  full(1, 3 * H), full(1, 3 * H), full(1, 1),
                  full(3 * H, 256), full(1, 256),
                  full(256, 256), full(1, 256),
                  full(256, D), full(1, D)],
        out_specs=pl.BlockSpec((BR, D), lambda i: (i, 0)),
        out_shape=jax.ShapeDtypeStruct((N, D), jnp.float32),
    )(xs0, xs1, xs2, x, dw, db, da, lw1, lb1, lw2, lb2, lw3, lb3)


def _tc_ab_jvec(z, w1a, w1b, jw1, jb1, jw2, jb2):
    def body(z_ref, wa_ref, wb_ref, jw1_ref, jb1_ref, jw2_ref, jb2_ref,
             a_ref, b_ref, jv_ref):
        zv = z_ref[...]
        a_ref[...] = jnp.dot(zv, wa_ref[...],
                             preferred_element_type=jnp.float32)
        b_ref[...] = jnp.dot(zv, wb_ref[...],
                             preferred_element_type=jnp.float32)
        gv = jnp.mean(zv, axis=0, keepdims=True)
        jv = jax.nn.gelu(jnp.dot(gv, jw1_ref[...],
                                 preferred_element_type=jnp.float32)
                         + jb1_ref[...])
        jv_ref[...] = jnp.dot(jv, jw2_ref[...],
                              preferred_element_type=jnp.float32) + jb2_ref[...]
    return pl.pallas_call(
        body,
        out_shape=[jax.ShapeDtypeStruct((N, D), jnp.float32),
                   jax.ShapeDtypeStruct((N, D), jnp.float32),
                   jax.ShapeDtypeStruct((1, D), jnp.float32)],
    )(z, w1a, w1b, jw1, jb1, jw2, jb2)


def _tc_contact(ga, gb, b1, w2, b2, w3r, b3):
    BE = 2000
    NBLK = E // BE

    def body(ga_ref, gb_ref, b1_ref, w2_ref, b2_ref, w3_ref, b3_ref, o_ref):
        c1 = jax.nn.gelu(ga_ref[...] + gb_ref[...] + b1_ref[...])
        c2 = jax.nn.gelu(jnp.dot(c1.astype(jnp.bfloat16),
                                 w2_ref[...].astype(jnp.bfloat16),
                                 preferred_element_type=jnp.float32)
                         + b2_ref[...])
        o = jnp.sum(c2 * w3_ref[...], axis=1, keepdims=True) + b3_ref[0, 0]
        o_ref[...] = jax.nn.sigmoid(o)

    blk = pl.BlockSpec((BE, D), lambda i: (i, 0))
    full = lambda r, c: pl.BlockSpec((r, c), lambda i: (0, 0))
    return pl.pallas_call(
        body,
        grid=(NBLK,),
        in_specs=[blk, blk, full(1, D), full(D, D), full(1, D),
                  full(1, D), full(1, 1)],
        out_specs=pl.BlockSpec((BE, 1), lambda i: (i, 0)),
        out_shape=jax.ShapeDtypeStruct((E, 1), jnp.float32),
    )(ga, gb, b1, w2, b2, w3r, b3)


def _row(v):
    return v.reshape(1, -1)


def kernel(x, edge_index, contact_pred_index, params):
    src, dst = edge_index[0], edge_index[1]
    cs, cd = contact_pred_index[0], contact_pred_index[1]
    pad = E_PAD - E
    zpad = jnp.zeros((pad,), jnp.int32)
    src_rows = jnp.concatenate([src, zpad]).reshape(IDX_ROWS, 128)
    dst_rows = jnp.concatenate(
        [dst, jnp.full((pad,), N, jnp.int32)]).reshape(IDX_ROWS, 128)
    cs_rows = jnp.concatenate([cs, zpad]).reshape(IDX_ROWS, 128)
    cd_rows = jnp.concatenate([cd, zpad]).reshape(IDX_ROWS, 128)

    g, b = params['bn']
    (w1, b1), (w2, b2) = params['in2model']
    h = _tc_prelude(x, _row(g), _row(b), w1, _row(b1), w2, _row(b2))

    agg_deg = _sc_agg(True)
    agg_only = _sc_agg(False)
    degp = None
    for i in range(3):
        ws, wn, bb = params['conv'][i]
        gw, gb_, ga = params['gn'][i]
        if i == 0:
            aggp, degp3 = agg_deg(h, src_rows, dst_rows)
            degp = degp3.reshape(NW, N_ACC)
        else:
            (aggp,) = agg_only(h, src_rows, dst_rows)
        h = _tc_layer(h, aggp, degp, ws, wn, _row(bb), _row(gw), _row(gb_),
                      ga.reshape(1, 1))
        if i == 0:
            xs0 = h
        elif i == 1:
            xs1 = h
        else:
            xs2 = h

    dw, db_, da = params['dyt']
    (lw1, lb1), (lw2, lb2), (lw3, lb3) = params['lin']
    z = _tc_final(xs0, xs1, xs2, x, _row(dw), _row(db_), da.reshape(1, 1),
                  lw1, _row(lb1), lw2, _row(lb2), lw3, _row(lb3))

    (cw1, cb1), (cw2, cb2), (cw3, cb3) = params['contact']
    (jw1, jb1), (jw2, jb2) = params['jproj']
    a_tab, b_tab, jv = _tc_ab_jvec(z, cw1[:D], cw1[D:], jw1, _row(jb1),
                                   jw2, _row(jb2))

    ga_rows, gb_rows = _sc_gather2(a_tab, b_tab, cs_rows, cd_rows)
    contact = _tc_contact(ga_rows, gb_rows, _row(cb1), cw2, _row(cb2),
                          cw3.reshape(1, D), cb3.reshape(1, 1))
    return z, contact, jv.reshape(D)


# speedup vs baseline: 1.0359x; 1.0359x over previous
import dataclasses
import functools

import jax
import jax.numpy as jnp
from jax import lax
from jax.experimental import pallas as pl
from jax.experimental.pallas import tpu as pltpu
from jax.experimental.pallas import tpu_sc as plsc

N = 10000
E = 320000
D = 128
H = 128

NC = 2
NS = 16
NW = NC * NS
E_PAD = 327680
IDX_ROWS = E_PAD // 128
SPLIT0 = 144
SPLIT1 = (IDX_ROWS // NS) - SPLIT0
N_ACC = 10240
ROWS_PER_TILE = N_ACC // NS


def _mesh():
    return plsc.VectorSubcoreMesh(core_axis_name="c", subcore_axis_name="s",
                                  num_cores=2)


def _sc_params():
    cp = pltpu.CompilerParams()
    if "needs_layout_passes" in pltpu.CompilerParams.__dataclass_fields__:
        cp = dataclasses.replace(cp, needs_layout_passes=False)
    return cp


def _zero_fill(buf, nrows, ncol16):
    @pl.loop(0, nrows)
    def _(r):
        for j in range(ncol16):
            buf[r, pl.ds(j * 16, 16)] = jnp.zeros((16,), jnp.float32)


def _sc_agg(with_deg):
    out_types = [jax.ShapeDtypeStruct((NC, N_ACC, D), jnp.float32)]
    scratch = [
        pltpu.VMEM((1, 128), jnp.int32),
        pltpu.VMEM((1, 128), jnp.int32),
        pltpu.VMEM((1, 128), jnp.int32),
        pltpu.VMEM((1, 128), jnp.int32),
        pltpu.VMEM((128, D), jnp.float32),
        pltpu.VMEM((128, D), jnp.float32),
        pltpu.VMEM((16, 128), jnp.float32),
        pltpu.VMEM_SHARED((N_ACC, D), jnp.float32),
        pltpu.SemaphoreType.DMA,
        pltpu.SemaphoreType.DMA,
        pltpu.SemaphoreType.DMA,
        pltpu.SemaphoreType.DMA,
    ]
    if with_deg:
        out_types.append(
            jax.ShapeDtypeStruct((NW, N_ACC // 128, 128), jnp.float32))
        scratch.append(pltpu.VMEM((N_ACC // 128, 128), jnp.float32))

    def body(h_hbm, src_hbm, dst_hbm, *refs):
        if with_deg:
            (agg_hbm, deg_hbm, si0, si1, di0, di1, rw0, rw1, zbuf, acc,
             smi0, smi1, smg0, smg1, degloc) = refs
        else:
            (agg_hbm, si0, si1, di0, di1, rw0, rw1, zbuf, acc,
             smi0, smi1, smg0, smg1) = refs
        c = lax.axis_index("c")
        s = lax.axis_index("s")
        sidx = [si0, si1]
        didx = [di0, di1]
        rows = [rw0, rw1]
        semi = [smi0, smi1]
        semg = [smg0, smg1]

        _zero_fill(zbuf, 16, 8)
        base = s * ROWS_PER_TILE
        @pl.loop(0, ROWS_PER_TILE // 16)
        def _(i):
            pltpu.sync_copy(zbuf, acc.at[pl.ds(base + i * 16, 16)])
        if with_deg:
            _zero_fill(degloc, N_ACC // 128, 8)
        plsc.subcore_barrier()

        ones16 = jnp.ones((16,), jnp.float32)

        def fire_idx(r, p):
            pltpu.async_copy(src_hbm.at[pl.ds(r, 1)], sidx[p], semi[p])
            pltpu.async_copy(dst_hbm.at[pl.ds(r, 1)], didx[p], semi[p])

        def wait_idx(r, p):
            pltpu.make_async_copy(src_hbm.at[pl.ds(r, 1)], sidx[p],
                                  semi[p]).wait()
            pltpu.make_async_copy(dst_hbm.at[pl.ds(r, 1)], didx[p],
                                  semi[p]).wait()

        def fire_gather(p):
            pltpu.async_copy(h_hbm.at[sidx[p].at[0]], rows[p], semg[p])

        def wait_gather(p):
            pltpu.make_async_copy(h_hbm.at[sidx[p].at[0]], rows[p],
                                  semg[p]).wait()

        def scatter(p):
            pltpu.sync_copy(rows[p], acc.at[didx[p].at[0]], add=True)
            if with_deg:
                for g in range(8):
                    dv = didx[p][0, pl.ds(g * 16, 16)]
                    plsc.addupdate_scatter(
                        degloc,
                        [lax.shift_right_logical(dv, 7),
                         lax.bitwise_and(dv, 127)],
                        ones16)

        def pipeline(nrows, wbase):
            fire_idx(wbase, 0)
            fire_idx(wbase + 1, 1)

            def chunk(u, p):
                wait_idx(wbase + u, p)
                fire_gather(p)
                @pl.when(u > 0)
                def _():
                    wait_gather(1 - p)
                    scatter(1 - p)
                    @pl.when(u < nrows - 1)
                    def _():
                        fire_idx(wbase + u + 1, 1 - p)

            @pl.loop(0, nrows, step=2)
            def _(t):
                chunk(t, 0)
                chunk(t + 1, 1)

            wait_gather(1)
            scatter(1)

        @pl.when(c == 0)
        def _():
            pipeline(SPLIT0, s * SPLIT0)

        @pl.when(c == 1)
        def _():
            pipeline(SPLIT1, NS * SPLIT0 + s * SPLIT1)

        plsc.subcore_barrier()
        pltpu.sync_copy(acc.at[pl.ds(base, ROWS_PER_TILE)],
                        agg_hbm.at[c].at[pl.ds(base, ROWS_PER_TILE)])
        if with_deg:
            pltpu.sync_copy(degloc, deg_hbm.at[c * NS + s])

    return pl.kernel(body, out_type=out_types, mesh=_mesh(),
                     scratch_types=scratch, compiler_params=_sc_params())


def _sc_gather2(tab_a, tab_b, idx_a, idx_b):
    def body(ta_hbm, tb_hbm, ia_hbm, ib_hbm, oa_hbm, ob_hbm,
             ba0, ba1, bb0, bb1, ra0, ra1, rb0, rb1,
             smi0, smi1, sma0, sma1, smb0, smb1, swa0, swa1, swb0, swb1):
        c = lax.axis_index("c")
        s = lax.axis_index("s")
        bia = [ba0, ba1]
        bib = [bb0, bb1]
        rowsa = [ra0, ra1]
        rowsb = [rb0, rb1]
        semi = [smi0, smi1]
        sema = [sma0, sma1]
        semb = [smb0, smb1]
        semwa = [swa0, swa1]
        semwb = [swb0, swb1]

        def fire_idx(r, p):
            pltpu.async_copy(ia_hbm.at[pl.ds(r, 1)], bia[p], semi[p])
            pltpu.async_copy(ib_hbm.at[pl.ds(r, 1)], bib[p], semi[p])

        def wait_idx(r, p):
            pltpu.make_async_copy(ia_hbm.at[pl.ds(r, 1)], bia[p],
                                  semi[p]).wait()
            pltpu.make_async_copy(ib_hbm.at[pl.ds(r, 1)], bib[p],
                                  semi[p]).wait()

        def fire_gather(p):
            pltpu.async_copy(ta_hbm.at[bia[p].at[0]], rowsa[p], sema[p])
            pltpu.async_copy(tb_hbm.at[bib[p].at[0]], rowsb[p], semb[p])

        def wait_gather(p):
            pltpu.make_async_copy(ta_hbm.at[bia[p].at[0]], rowsa[p],
                                  sema[p]).wait()
            pltpu.make_async_copy(tb_hbm.at[bib[p].at[0]], rowsb[p],
                                  semb[p]).wait()

        def fire_write(r, p):
            pltpu.async_copy(rowsa[p], oa_hbm.at[pl.ds(r * 128, 128)],
                             semwa[p])
            pltpu.async_copy(rowsb[p], ob_hbm.at[pl.ds(r * 128, 128)],
                             semwb[p])

        def wait_write(r, p):
            pltpu.make_async_copy(rowsa[p], oa_hbm.at[pl.ds(r * 128, 128)],
                                  semwa[p]).wait()
            pltpu.make_async_copy(rowsb[p], ob_hbm.at[pl.ds(r * 128, 128)],
                                  semwb[p]).wait()

        def pipeline(nrows, wbase):
            fire_idx(wbase, 0)
            fire_idx(wbase + 1, 1)

            def chunk(u, p):
                wait_idx(wbase + u, p)
                @pl.when(u > 1)
                def _():
                    wait_write(wbase + u - 2, p)
                fire_gather(p)
                @pl.when(u > 0)
                def _():
                    wait_gather(1 - p)
                    fire_write(wbase + u - 1, 1 - p)
                    @pl.when(u < nrows - 1)
                    def _():
                        fire_idx(wbase + u + 1, 1 - p)

            @pl.loop(0, nrows, step=2)
            def _(t):
                chunk(t, 0)
                chunk(t + 1, 1)

            wait_gather(1)
            fire_write(wbase + nrows - 1, 1)
            wait_write(wbase + nrows - 2, 0)
            wait_write(wbase + nrows - 1, 1)

        @pl.when(c == 0)
        def _():
            pipeline(SPLIT0, s * SPLIT0)

        @pl.when(c == 1)
        def _():
            pipeline(SPLIT1, NS * SPLIT0 + s * SPLIT1)

    k = pl.kernel(
        body,
        out_type=[jax.ShapeDtypeStruct((E_PAD, D), jnp.float32),
                  jax.ShapeDtypeStruct((E_PAD, D), jnp.float32)],
        mesh=_mesh(),
        scratch_types=(
            [pltpu.VMEM((1, 128), jnp.int32) for _ in range(4)]
            + [pltpu.VMEM((128, D), jnp.float32) for _ in range(4)]
            + [pltpu.SemaphoreType.DMA for _ in range(10)]
        ),
        compiler_params=_sc_params(),
    )
    return k(tab_a, tab_b, idx_a, idx_b)


def _tc_prelude(x, g, b, w1, b1, w2, b2):
    def body(x_ref, g_ref, b_ref, w1_ref, b1_ref, w2_ref, b2_ref, o_ref):
        xv = x_ref[...]
        mu = jnp.mean(xv, axis=0, keepdims=True)
        var = jnp.mean((xv - mu) ** 2, axis=0, keepdims=True)
        h = (xv - mu) / jnp.sqrt(var + 1e-5) * g_ref[...] + b_ref[...]
        h = jax.nn.gelu(jnp.dot(h, w1_ref[...],
                                preferred_element_type=jnp.float32)
                        + b1_ref[...])
        h = jnp.tanh(jnp.dot(h, w2_ref[...],
                             preferred_element_type=jnp.float32) + b2_ref[...])
        o_ref[...] = h
    return pl.pallas_call(
        body, out_shape=jax.ShapeDtypeStruct((N, D), jnp.float32),
    )(x, g, b, w1, b1, w2, b2)


def _tc_layer(h, agg, degp, ws, wn, bb, gw, gb, ga):
    def body(h_ref, a_ref, d_ref, ws_ref, wn_ref, bb_ref, gw_ref, gb_ref,
             ga_ref, o_ref):
        deg = lax.dot_general(
            d_ref[...], jnp.ones((NW, 1), jnp.float32),
            (((0,), (0,)), ((), ())),
            preferred_element_type=jnp.float32)[:N]
        deg = jnp.maximum(deg, 1.0)
        agg_v = (a_ref[0, :N, :] + a_ref[1, :N, :]) / deg
        hv = h_ref[...]
        h_new = (jnp.dot(hv, ws_ref[...], preferred_element_type=jnp.float32)
                 + jnp.dot(agg_v, wn_ref[...],
                           preferred_element_type=jnp.float32)
                 + bb_ref[...])
        m = jnp.mean(h_new, axis=0, keepdims=True)
        xc = h_new - ga_ref[0, 0] * m
        v = jnp.mean(xc ** 2, axis=0, keepdims=True)
        o_ref[...] = gw_ref[...] * xc / jnp.sqrt(v + 1e-5) + gb_ref[...]
    return pl.pallas_call(
        body, out_shape=jax.ShapeDtypeStruct((N, H), jnp.float32),
    )(h, agg, degp, ws, wn, bb, gw, gb, ga)


def _tc_final(xs0, xs1, xs2, x, dw, db, da, lw1, lb1, lw2, lb2, lw3, lb3):
    NB = 5
    BR = N // NB

    def body(x0_ref, x1_ref, x2_ref, x_ref, dw_ref, db_ref, da_ref,
             w1_ref, b1_ref, w2_ref, b2_ref, w3_ref, b3_ref, o_ref):
        hcat = jnp.concatenate(
            [x0_ref[...], x1_ref[...], x2_ref[...]], axis=1)
        t = dw_ref[...] * jnp.tanh(da_ref[0, 0] * hcat) + db_ref[...]
        t = jax.nn.gelu(jnp.dot(t, w1_ref[...],
                                preferred_element_type=jnp.float32)
                        + b1_ref[...])
        t = jax.nn.gelu(jnp.dot(t, w2_ref[...],
                                preferred_element_type=jnp.float32)
                        + b2_ref[...])
        t = jnp.dot(t, w3_ref[...],
                    preferred_element_type=jnp.float32) + b3_ref[...]
        o_ref[...] = t + x_ref[...]

    row_spec = pl.BlockSpec((BR, H), lambda i: (i, 0))
    full = lambda r, c: pl.BlockSpec((r, c), lambda i: (0, 0))
    return pl.pallas_call(
        body,
        grid=(NB,),
        in_specs=[row_spec, row_spec, row_spec, row_spec,
                  full(1, 3 * H), full(1, 3 * H), full(1, 1),
                  full(3 * H, 256), full(1, 256),
                  full(256, 256), full(1, 256),
                  full(256, D), full(1, D)],
        out_specs=pl.BlockSpec((BR, D), lambda i: (i, 0)),
        out_shape=jax.ShapeDtypeStruct((N, D), jnp.float32),
    )(xs0, xs1, xs2, x, dw, db, da, lw1, lb1, lw2, lb2, lw3, lb3)


def _tc_ab_jvec(z, w1a, w1b, jw1, jb1, jw2, jb2):
    def body(z_ref, wa_ref, wb_ref, jw1_ref, jb1_ref, jw2_ref, jb2_ref,
             a_ref, b_ref, jv_ref):
        zv = z_ref[...]
        a_ref[...] = jnp.dot(zv, wa_ref[...],
                             preferred_element_type=jnp.float32)
        b_ref[...] = jnp.dot(zv, wb_ref[...],
                             preferred_element_type=jnp.float32)
        gv = jnp.mean(zv, axis=0, keepdims=True)
        jv = jax.nn.gelu(jnp.dot(gv, jw1_ref[...],
                                 preferred_element_type=jnp.float32)
                         + jb1_ref[...])
        jv_ref[...] = jnp.dot(jv, jw2_ref[...],
                              preferred_element_type=jnp.float32) + jb2_ref[...]
    return pl.pallas_call(
        body,
        out_shape=[jax.ShapeDtypeStruct((N, D), jnp.float32),
                   jax.ShapeDtypeStruct((N, D), jnp.float32),
                   jax.ShapeDtypeStruct((1, D), jnp.float32)],
    )(z, w1a, w1b, jw1, jb1, jw2, jb2)


def _tc_contact(ga, gb, b1, w2, b2, w3r, b3):
    BE = 2000
    NBLK = E // BE

    def body(ga_ref, gb_ref, b1_ref, w2_ref, b2_ref, w3_ref, b3_ref, o_ref):
        c1 = jax.nn.gelu(ga_ref[...] + gb_ref[...] + b1_ref[...])
        c2 = jax.nn.gelu(jnp.dot(c1.astype(jnp.bfloat16),
                                 w2_ref[...].astype(jnp.bfloat16),
                                 preferred_element_type=jnp.float32)
                         + b2_ref[...])
        o = jnp.sum(c2 * w3_ref[...], axis=1, keepdims=True) + b3_ref[0, 0]
        o_ref[...] = jax.nn.sigmoid(o)

    blk = pl.BlockSpec((BE, D), lambda i: (i, 0))
    full = lambda r, c: pl.BlockSpec((r, c), lambda i: (0, 0))
    return pl.pallas_call(
        body,
        grid=(NBLK,),
        in_specs=[blk, blk, full(1, D), full(D, D), full(1, D),
                  full(1, D), full(1, 1)],
        out_specs=pl.BlockSpec((BE, 1), lambda i: (i, 0)),
        out_shape=jax.ShapeDtypeStruct((E, 1), jnp.float32),
    )(ga, gb, b1, w2, b2, w3r, b3)


def _row(v):
    return v.reshape(1, -1)


def kernel(x, edge_index, contact_pred_index, params):
    src, dst = edge_index[0], edge_index[1]
    cs, cd = contact_pred_index[0], contact_pred_index[1]
    pad = E_PAD - E
    zpad = jnp.zeros((pad,), jnp.int32)
    src_rows = jnp.concatenate([src, zpad]).reshape(IDX_ROWS, 128)
    dst_rows = jnp.concatenate(
        [dst, jnp.full((pad,), N, jnp.int32)]).reshape(IDX_ROWS, 128)
    cs_rows = jnp.concatenate([cs, zpad]).reshape(IDX_ROWS, 128)
    cd_rows = jnp.concatenate([cd, zpad]).reshape(IDX_ROWS, 128)

    g, b = params['bn']
    (w1, b1), (w2, b2) = params['in2model']
    h = _tc_prelude(x, _row(g), _row(b), w1, _row(b1), w2, _row(b2))

    agg_deg = _sc_agg(True)
    agg_only = _sc_agg(False)
    degp = None
    for i in range(3):
        ws, wn, bb = params['conv'][i]
        gw, gb_, ga = params['gn'][i]
        if i == 0:
            aggp, degp3 = agg_deg(h, src_rows, dst_rows)
            degp = degp3.reshape(NW, N_ACC)
        else:
            (aggp,) = agg_only(h, src_rows, dst_rows)
        h = _tc_layer(h, aggp, degp, ws, wn, _row(bb), _row(gw), _row(gb_),
                      ga.reshape(1, 1))
        if i == 0:
            xs0 = h
        elif i == 1:
            xs1 = h
        else:
            xs2 = h

    dw, db_, da = params['dyt']
    (lw1, lb1), (lw2, lb2), (lw3, lb3) = params['lin']
    z = _tc_final(xs0, xs1, xs2, x, _row(dw), _row(db_), da.reshape(1, 1),
                  lw1, _row(lb1), lw2, _row(lb2), lw3, _row(lb3))

    (cw1, cb1), (cw2, cb2), (cw3, cb3) = params['contact']
    (jw1, jb1), (jw2, jb2) = params['jproj']
    a_tab, b_tab, jv = _tc_ab_jvec(z, cw1[:D], cw1[D:], jw1, _row(jb1),
                                   jw2, _row(jb2))

    ga_rows, gb_rows = _sc_gather2(a_tab, b_tab, cs_rows, cd_rows)
    contact = _tc_contact(ga_rows, gb_rows, _row(cb1), cw2, _row(cb2),
                          cw3.reshape(1, D), cb3.reshape(1, 1))
    return z, contact, jv.reshape(D)


# speedup vs baseline: 1.0439x; 1.0077x over previous
import dataclasses
import functools

import jax
import jax.numpy as jnp
from jax import lax
from jax.experimental import pallas as pl
from jax.experimental.pallas import tpu as pltpu
from jax.experimental.pallas import tpu_sc as plsc

N = 10000
E = 320000
D = 128
H = 128

NC = 2
NS = 16
NW = NC * NS
E_PAD = 327680
IDX_ROWS = E_PAD // 128
SPLIT0 = 144
SPLIT1 = (IDX_ROWS // NS) - SPLIT0
N_ACC = 10240
ROWS_PER_TILE = N_ACC // NS


def _mesh():
    return plsc.VectorSubcoreMesh(core_axis_name="c", subcore_axis_name="s",
                                  num_cores=2)


def _sc_params():
    cp = pltpu.CompilerParams()
    if "needs_layout_passes" in pltpu.CompilerParams.__dataclass_fields__:
        cp = dataclasses.replace(cp, needs_layout_passes=False)
    return cp


def _zero_fill(buf, nrows, ncol16):
    @pl.loop(0, nrows)
    def _(r):
        for j in range(ncol16):
            buf[r, pl.ds(j * 16, 16)] = jnp.zeros((16,), jnp.float32)


def _sc_agg(with_deg):
    out_types = [jax.ShapeDtypeStruct((NC, N_ACC, D), jnp.float32)]
    scratch = [
        pltpu.VMEM((1, 128), jnp.int32),
        pltpu.VMEM((1, 128), jnp.int32),
        pltpu.VMEM((1, 128), jnp.int32),
        pltpu.VMEM((1, 128), jnp.int32),
        pltpu.VMEM((128, D), jnp.float32),
        pltpu.VMEM((128, D), jnp.float32),
        pltpu.VMEM((16, 128), jnp.float32),
        pltpu.VMEM_SHARED((N_ACC, D), jnp.float32),
        pltpu.SemaphoreType.DMA,
        pltpu.SemaphoreType.DMA,
        pltpu.SemaphoreType.DMA,
        pltpu.SemaphoreType.DMA,
    ]
    if with_deg:
        out_types.append(
            jax.ShapeDtypeStruct((NW, N_ACC // 128, 128), jnp.float32))
        scratch.append(pltpu.VMEM((N_ACC // 128, 128), jnp.float32))

    def body(h_hbm, src_hbm, dst_hbm, *refs):
        if with_deg:
            (agg_hbm, deg_hbm, si0, si1, di0, di1, rw0, rw1, zbuf, acc,
             smi0, smi1, smg0, smg1, degloc) = refs
        else:
            (agg_hbm, si0, si1, di0, di1, rw0, rw1, zbuf, acc,
             smi0, smi1, smg0, smg1) = refs
        c = lax.axis_index("c")
        s = lax.axis_index("s")
        sidx = [si0, si1]
        didx = [di0, di1]
        rows = [rw0, rw1]
        semi = [smi0, smi1]
        semg = [smg0, smg1]

        _zero_fill(zbuf, 16, 8)
        base = s * ROWS_PER_TILE
        @pl.loop(0, ROWS_PER_TILE // 16)
        def _(i):
            pltpu.sync_copy(zbuf, acc.at[pl.ds(base + i * 16, 16)])
        if with_deg:
            _zero_fill(degloc, N_ACC // 128, 8)
        plsc.subcore_barrier()

        ones16 = jnp.ones((16,), jnp.float32)

        def fire_idx(r, p):
            pltpu.async_copy(src_hbm.at[pl.ds(r, 1)], sidx[p], semi[p])
            pltpu.async_copy(dst_hbm.at[pl.ds(r, 1)], didx[p], semi[p])

        def wait_idx(r, p):
            pltpu.make_async_copy(src_hbm.at[pl.ds(r, 1)], sidx[p],
                                  semi[p]).wait()
            pltpu.make_async_copy(dst_hbm.at[pl.ds(r, 1)], didx[p],
                                  semi[p]).wait()

        def fire_gather(p):
            pltpu.async_copy(h_hbm.at[sidx[p].at[0]], rows[p], semg[p])

        def wait_gather(p):
            pltpu.make_async_copy(h_hbm.at[sidx[p].at[0]], rows[p],
                                  semg[p]).wait()

        def scatter(p):
            pltpu.sync_copy(rows[p], acc.at[didx[p].at[0]], add=True)
            if with_deg:
                for g in range(8):
                    dv = didx[p][0, pl.ds(g * 16, 16)]
                    plsc.addupdate_scatter(
                        degloc,
                        [lax.shift_right_logical(dv, 7),
                         lax.bitwise_and(dv, 127)],
                        ones16)

        def pipeline(nrows, wbase):
            fire_idx(wbase, 0)
            fire_idx(wbase + 1, 1)

            def chunk(u, p):
                wait_idx(wbase + u, p)
                fire_gather(p)
                @pl.when(u > 0)
                def _():
                    wait_gather(1 - p)
                    scatter(1 - p)
                    @pl.when(u < nrows - 1)
                    def _():
                        fire_idx(wbase + u + 1, 1 - p)

            @pl.loop(0, nrows, step=2)
            def _(t):
                chunk(t, 0)
                chunk(t + 1, 1)

            wait_gather(1)
            scatter(1)

        @pl.when(c == 0)
        def _():
            pipeline(SPLIT0, s * SPLIT0)

        @pl.when(c == 1)
        def _():
            pipeline(SPLIT1, NS * SPLIT0 + s * SPLIT1)

        plsc.subcore_barrier()
        pltpu.sync_copy(acc.at[pl.ds(base, ROWS_PER_TILE)],
                        agg_hbm.at[c].at[pl.ds(base, ROWS_PER_TILE)])
        if with_deg:
            pltpu.sync_copy(degloc, deg_hbm.at[c * NS + s])

    return pl.kernel(body, out_type=out_types, mesh=_mesh(),
                     scratch_types=scratch, compiler_params=_sc_params())


def _sc_gather2(tab_a, tab_b, idx_a, idx_b):
    def body(ta_hbm, tb_hbm, ia_hbm, ib_hbm, oa_hbm, ob_hbm,
             ba0, ba1, bb0, bb1, ra0, ra1, rb0, rb1,
             smi0, smi1, sma0, sma1, smb0, smb1, swa0, swa1, swb0, swb1):
        c = lax.axis_index("c")
        s = lax.axis_index("s")
        bia = [ba0, ba1]
        bib = [bb0, bb1]
        rowsa = [ra0, ra1]
        rowsb = [rb0, rb1]
        semi = [smi0, smi1]
        sema = [sma0, sma1]
        semb = [smb0, smb1]
        semwa = [swa0, swa1]
        semwb = [swb0, swb1]

        def fire_idx(r, p):
            pltpu.async_copy(ia_hbm.at[pl.ds(r, 1)], bia[p], semi[p])
            pltpu.async_copy(ib_hbm.at[pl.ds(r, 1)], bib[p], semi[p])

        def wait_idx(r, p):
            pltpu.make_async_copy(ia_hbm.at[pl.ds(r, 1)], bia[p],
                                  semi[p]).wait()
            pltpu.make_async_copy(ib_hbm.at[pl.ds(r, 1)], bib[p],
                                  semi[p]).wait()

        def fire_gather(p):
            pltpu.async_copy(ta_hbm.at[bia[p].at[0]], rowsa[p], sema[p])
            pltpu.async_copy(tb_hbm.at[bib[p].at[0]], rowsb[p], semb[p])

        def wait_gather(p):
            pltpu.make_async_copy(ta_hbm.at[bia[p].at[0]], rowsa[p],
                                  sema[p]).wait()
            pltpu.make_async_copy(tb_hbm.at[bib[p].at[0]], rowsb[p],
                                  semb[p]).wait()

        def fire_write(r, p):
            pltpu.async_copy(rowsa[p], oa_hbm.at[pl.ds(r * 128, 128)],
                             semwa[p])
            pltpu.async_copy(rowsb[p], ob_hbm.at[pl.ds(r * 128, 128)],
                             semwb[p])

        def wait_write(r, p):
            pltpu.make_async_copy(rowsa[p], oa_hbm.at[pl.ds(r * 128, 128)],
                                  semwa[p]).wait()
            pltpu.make_async_copy(rowsb[p], ob_hbm.at[pl.ds(r * 128, 128)],
                                  semwb[p]).wait()

        def pipeline(nrows, wbase):
            fire_idx(wbase, 0)
            fire_idx(wbase + 1, 1)

            def chunk(u, p):
                wait_idx(wbase + u, p)
                @pl.when(u > 1)
                def _():
                    wait_write(wbase + u - 2, p)
                fire_gather(p)
                @pl.when(u > 0)
                def _():
                    wait_gather(1 - p)
                    fire_write(wbase + u - 1, 1 - p)
                    @pl.when(u < nrows - 1)
                    def _():
                        fire_idx(wbase + u + 1, 1 - p)

            @pl.loop(0, nrows, step=2)
            def _(t):
                chunk(t, 0)
                chunk(t + 1, 1)

            wait_gather(1)
            fire_write(wbase + nrows - 1, 1)
            wait_write(wbase + nrows - 2, 0)
            wait_write(wbase + nrows - 1, 1)

        @pl.when(c == 0)
        def _():
            pipeline(SPLIT0, s * SPLIT0)

        @pl.when(c == 1)
        def _():
            pipeline(SPLIT1, NS * SPLIT0 + s * SPLIT1)

    k = pl.kernel(
        body,
        out_type=[jax.ShapeDtypeStruct((E_PAD, D), jnp.float32),
                  jax.ShapeDtypeStruct((E_PAD, D), jnp.float32)],
        mesh=_mesh(),
        scratch_types=(
            [pltpu.VMEM((1, 128), jnp.int32) for _ in range(4)]
            + [pltpu.VMEM((128, D), jnp.float32) for _ in range(4)]
            + [pltpu.SemaphoreType.DMA for _ in range(10)]
        ),
        compiler_params=_sc_params(),
    )
    return k(tab_a, tab_b, idx_a, idx_b)


def _tc_prelude(x, g, b, w1, b1, w2, b2):
    def body(x_ref, g_ref, b_ref, w1_ref, b1_ref, w2_ref, b2_ref, o_ref):
        xv = x_ref[...]
        mu = jnp.mean(xv, axis=0, keepdims=True)
        var = jnp.mean((xv - mu) ** 2, axis=0, keepdims=True)
        h = (xv - mu) / jnp.sqrt(var + 1e-5) * g_ref[...] + b_ref[...]
        h = jax.nn.gelu(jnp.dot(h, w1_ref[...],
                                preferred_element_type=jnp.float32)
                        + b1_ref[...])
        h = jnp.tanh(jnp.dot(h, w2_ref[...],
                             preferred_element_type=jnp.float32) + b2_ref[...])
        o_ref[...] = h
    return pl.pallas_call(
        body, out_shape=jax.ShapeDtypeStruct((N, D), jnp.float32),
    )(x, g, b, w1, b1, w2, b2)


def _tc_layer(h, agg, degp, ws, wn, bb, gw, gb, ga):
    def body(h_ref, a_ref, d_ref, ws_ref, wn_ref, bb_ref, gw_ref, gb_ref,
             ga_ref, o_ref):
        deg = lax.dot_general(
            d_ref[...], jnp.ones((NW, 1), jnp.float32),
            (((0,), (0,)), ((), ())),
            preferred_element_type=jnp.float32)[:N]
        deg = jnp.maximum(deg, 1.0)
        agg_v = (a_ref[0, :N, :] + a_ref[1, :N, :]) / deg
        hv = h_ref[...]
        h_new = (jnp.dot(hv, ws_ref[...], preferred_element_type=jnp.float32)
                 + jnp.dot(agg_v, wn_ref[...],
                           preferred_element_type=jnp.float32)
                 + bb_ref[...])
        m = jnp.mean(h_new, axis=0, keepdims=True)
        xc = h_new - ga_ref[0, 0] * m
        v = jnp.mean(xc ** 2, axis=0, keepdims=True)
        o_ref[...] = gw_ref[...] * xc / jnp.sqrt(v + 1e-5) + gb_ref[...]
    return pl.pallas_call(
        body, out_shape=jax.ShapeDtypeStruct((N, H), jnp.float32),
    )(h, agg, degp, ws, wn, bb, gw, gb, ga)


def _tc_final(xs0, xs1, xs2, x, dw, db, da, lw1, lb1, lw2, lb2, lw3, lb3):
    NB = 5
    BR = N // NB

    def body(x0_ref, x1_ref, x2_ref, x_ref, dw_ref, db_ref, da_ref,
             w1_ref, b1_ref, w2_ref, b2_ref, w3_ref, b3_ref, o_ref):
        hcat = jnp.concatenate(
            [x0_ref[...], x1_ref[...], x2_ref[...]], axis=1)
        t = dw_ref[...] * jnp.tanh(da_ref[0, 0] * hcat) + db_ref[...]
        t = jax.nn.gelu(jnp.dot(t, w1_ref[...],
                                preferred_element_type=jnp.float32)
                        + b1_ref[...])
        t = jax.nn.gelu(jnp.dot(t, w2_ref[...],
                                preferred_element_type=jnp.float32)
                        + b2_ref[...])
        t = jnp.dot(t, w3_ref[...],
                    preferred_element_type=jnp.float32) + b3_ref[...]
        o_ref[...] = t + x_ref[...]

    row_spec = pl.BlockSpec((BR, H), lambda i: (i, 0))
    full = lambda r, c: pl.BlockSpec((r, c), lambda i: (0, 0))
    return pl.pallas_call(
        body,
        grid=(NB,),
        in_specs=[row_spec, row_spec, row_spec, row_spec,
                  full(1, 3 * H), full(1, 3 * H), full(1, 1),
                  full(3 * H, 256), full(1, 256),
                  full(256, 256), full(1, 256),
                  full(256, D), full(1, D)],
        out_specs=pl.BlockSpec((BR, D), lambda i: (i, 0)),
        out_shape=jax.ShapeDtypeStruct((N, D), jnp.float32),
    )(xs0, xs1, xs2, x, dw, db, da, lw1, lb1, lw2, lb2, lw3, lb3)


def _tc_ab_jvec(z, w1a, w1b, jw1, jb1, jw2, jb2):
    def body(z_ref, wa_ref, wb_ref, jw1_ref, jb1_ref, jw2_ref, jb2_ref,
             a_ref, b_ref, jv_ref):
        zv = z_ref[...]
        a_ref[...] = jnp.dot(zv, wa_ref[...],
                             preferred_element_type=jnp.float32)
        b_ref[...] = jnp.dot(zv, wb_ref[...],
                             preferred_element_type=jnp.float32)
        gv = jnp.mean(zv, axis=0, keepdims=True)
        jv = jax.nn.gelu(jnp.dot(gv, jw1_ref[...],
                                 preferred_element_type=jnp.float32)
                         + jb1_ref[...])
        jv_ref[...] = jnp.dot(jv, jw2_ref[...],
                              preferred_element_type=jnp.float32) + jb2_ref[...]
    return pl.pallas_call(
        body,
        out_shape=[jax.ShapeDtypeStruct((N, D), jnp.float32),
                   jax.ShapeDtypeStruct((N, D), jnp.float32),
                   jax.ShapeDtypeStruct((1, D), jnp.float32)],
    )(z, w1a, w1b, jw1, jb1, jw2, jb2)


def _tc_contact(ga, gb, b1, w2, b2, w3r, b3):
    BE = 2000
    NBLK = E // BE

    def body(ga_ref, gb_ref, b1_ref, w2_ref, b2_ref, w3_ref, b3_ref, o_ref):
        p1 = (ga_ref[...] + gb_ref[...] + b1_ref[...]).astype(jnp.bfloat16)
        c1 = jax.nn.gelu(p1)
        p2 = (jnp.dot(c1, w2_ref[...].astype(jnp.bfloat16),
                      preferred_element_type=jnp.float32)
              + b2_ref[...]).astype(jnp.bfloat16)
        c2 = jax.nn.gelu(p2).astype(jnp.float32)
        o = jnp.sum(c2 * w3_ref[...], axis=1, keepdims=True) + b3_ref[0, 0]
        o_ref[...] = jax.nn.sigmoid(o)

    blk = pl.BlockSpec((BE, D), lambda i: (i, 0))
    full = lambda r, c: pl.BlockSpec((r, c), lambda i: (0, 0))
    return pl.pallas_call(
        body,
        grid=(NBLK,),
        in_specs=[blk, blk, full(1, D), full(D, D), full(1, D),
                  full(1, D), full(1, 1)],
        out_specs=pl.BlockSpec((BE, 1), lambda i: (i, 0)),
        out_shape=jax.ShapeDtypeStruct((E, 1), jnp.float32),
    )(ga, gb, b1, w2, b2, w3r, b3)


def _row(v):
    return v.reshape(1, -1)


def kernel(x, edge_index, contact_pred_index, params):
    src, dst = edge_index[0], edge_index[1]
    cs, cd = contact_pred_index[0], contact_pred_index[1]
    pad = E_PAD - E
    zpad = jnp.zeros((pad,), jnp.int32)
    src_rows = jnp.concatenate([src, zpad]).reshape(IDX_ROWS, 128)
    dst_rows = jnp.concatenate(
        [dst, jnp.full((pad,), N, jnp.int32)]).reshape(IDX_ROWS, 128)
    cs_rows = jnp.concatenate([cs, zpad]).reshape(IDX_ROWS, 128)
    cd_rows = jnp.concatenate([cd, zpad]).reshape(IDX_ROWS, 128)

    g, b = params['bn']
    (w1, b1), (w2, b2) = params['in2model']
    h = _tc_prelude(x, _row(g), _row(b), w1, _row(b1), w2, _row(b2))

    agg_deg = _sc_agg(True)
    agg_only = _sc_agg(False)
    degp = None
    for i in range(3):
        ws, wn, bb = params['conv'][i]
        gw, gb_, ga = params['gn'][i]
        if i == 0:
            aggp, degp3 = agg_deg(h, src_rows, dst_rows)
            degp = degp3.reshape(NW, N_ACC)
        else:
            (aggp,) = agg_only(h, src_rows, dst_rows)
        h = _tc_layer(h, aggp, degp, ws, wn, _row(bb), _row(gw), _row(gb_),
                      ga.reshape(1, 1))
        if i == 0:
            xs0 = h
        elif i == 1:
            xs1 = h
        else:
            xs2 = h

    dw, db_, da = params['dyt']
    (lw1, lb1), (lw2, lb2), (lw3, lb3) = params['lin']
    z = _tc_final(xs0, xs1, xs2, x, _row(dw), _row(db_), da.reshape(1, 1),
                  lw1, _row(lb1), lw2, _row(lb2), lw3, _row(lb3))

    (cw1, cb1), (cw2, cb2), (cw3, cb3) = params['contact']
    (jw1, jb1), (jw2, jb2) = params['jproj']
    a_tab, b_tab, jv = _tc_ab_jvec(z, cw1[:D], cw1[D:], jw1, _row(jb1),
                                   jw2, _row(jb2))

    ga_rows, gb_rows = _sc_gather2(a_tab, b_tab, cs_rows, cd_rows)
    contact = _tc_contact(ga_rows, gb_rows, _row(cb1), cw2, _row(cb2),
                          cw3.reshape(1, D), cb3.reshape(1, 1))
    return z, contact, jv.reshape(D)
